# Initial kernel scaffold; baseline (speedup 1.0000x reference)
import jax
import jax.numpy as jnp
from jax.experimental import pallas as pl


def kernel(z, codebook):
    raise NotImplementedError("write your pallas kernel here")


# speedup vs baseline: 1.1165x; 1.1165x over previous
import functools

import jax
import jax.numpy as jnp
from jax import lax
from jax.experimental import pallas as pl
from jax.experimental.pallas import tpu as pltpu
from jax.experimental.pallas import tpu_sc as plsc

_EMBED_DIM = 256
_N_CODES = 8192
_BATCH = 8
_TOKENS_PER_BATCH = 1024
_N_TOKENS = _BATCH * _TOKENS_PER_BATCH
_KBLK = 512
_N_KBLKS = _N_CODES // _KBLK
_BETA = 0.25


_WINDOWS = ((0, 4096), (4096, 8192))
_CHUNK = 1024


def _argmin_body(cb_ref, zr_ref, mi_ref):
    zb = zr_ref[0]
    h = zb * zb
    n = _EMBED_DIM
    while n > 1:
        n //= 2
        h = h[0:n, :] + h[n:2 * n, :]
    zn = h
    zb2 = (2.0 * zb).astype(jnp.bfloat16)
    carry_v = None
    carry_i = None
    for ws, we in _WINDOWS:
        wv = None
        wi = None
        s = ws
        while s < we:
            n = min(_CHUNK, we - s)
            cbc = cb_ref[pl.ds(s, n), :]
            cn = jnp.sum(cbc * cbc, axis=1, keepdims=True)
            mm = lax.dot_general(cbc.astype(jnp.bfloat16), zb2,
                                 (((1,), (0,)), ((), ())),
                                 preferred_element_type=jnp.float32)
            d = (zn + cn) - mm
            bmv = jnp.min(d, axis=0, keepdims=True)
            rows = lax.broadcasted_iota(jnp.int32, d.shape, 0)
            bmi = jnp.min(jnp.where(d == bmv, rows, jnp.int32(2**30)),
                          axis=0, keepdims=True) + s
            if wv is None:
                wv, wi = bmv, bmi
            else:
                take = bmv < wv
                wv = jnp.where(take, bmv, wv)
                wi = jnp.where(take, bmi, wi)
            s += n
        if carry_v is None:
            carry_v, carry_i = wv, wi
        else:
            take = wv < carry_v
            carry_v = jnp.where(take, wv, carry_v)
            carry_i = jnp.where(take, wi, carry_i)
        carry_v = carry_v.astype(jnp.bfloat16).astype(jnp.float32)
    mi_ref[0] = carry_i


def _argmin_call(codebook, zr):
    return pl.pallas_call(
        _argmin_body,
        grid=(_BATCH,),
        in_specs=[
            pl.BlockSpec((_N_CODES, _EMBED_DIM), lambda b: (0, 0)),
            pl.BlockSpec((1, _EMBED_DIM, _TOKENS_PER_BATCH),
                         lambda b: (b, 0, 0)),
        ],
        out_specs=[
            pl.BlockSpec((1, 1, _TOKENS_PER_BATCH), lambda b: (b, 0, 0)),
        ],
        out_shape=[
            jax.ShapeDtypeStruct((_BATCH, 1, _TOKENS_PER_BATCH), jnp.int32),
        ],
        compiler_params=pltpu.CompilerParams(
            dimension_semantics=("arbitrary",)),
    )(codebook, zr)


_SC_CORES = 2
_SC_SUBCORES = 16
_NW = _SC_CORES * _SC_SUBCORES
_ROWS_PER_W = _N_TOKENS // _NW
_GCHUNK = 128


@functools.cache
def _sc_gather_kernel():
    @functools.partial(
        pl.kernel,
        mesh=plsc.VectorSubcoreMesh(core_axis_name="c", subcore_axis_name="s"),
        out_type=jax.ShapeDtypeStruct((_N_TOKENS, _EMBED_DIM), jnp.float32),
        scratch_types=[
            pltpu.VMEM((_GCHUNK,), jnp.int32),
            pltpu.VMEM((_GCHUNK, _EMBED_DIM), jnp.float32),
            pltpu.SemaphoreType.DMA,
        ],
    )
    def _sc_gather(cb_hbm, idx_hbm, out_hbm, idx_v, rows_v, sem):
        wid = lax.axis_index("s") * _SC_CORES + lax.axis_index("c")
        base = wid * _ROWS_PER_W
        for c in range(_ROWS_PER_W // _GCHUNK):
            off = base + c * _GCHUNK
            pltpu.sync_copy(idx_hbm.at[pl.ds(off, _GCHUNK)], idx_v)
            pltpu.async_copy(cb_hbm.at[idx_v], rows_v, sem).wait()
            pltpu.sync_copy(rows_v, out_hbm.at[pl.ds(off, _GCHUNK)])

    return _sc_gather


def _assemble_body(zq_ref, zr_ref, out_ref, loss_ref):
    b = pl.program_id(0)
    zq_t = zq_ref[...].T
    zp = zr_ref[0]
    diff = zq_t - zp
    out_ref[0] = zp + diff
    part = jnp.sum(diff * diff)

    @pl.when(b == 0)
    def _():
        loss_ref[0, 0] = part

    @pl.when(b != 0)
    def _():
        loss_ref[0, 0] = loss_ref[0, 0] + part


def _assemble_call(zq, zr):
    return pl.pallas_call(
        _assemble_body,
        grid=(_BATCH,),
        in_specs=[
            pl.BlockSpec((_TOKENS_PER_BATCH, _EMBED_DIM), lambda b: (b, 0)),
            pl.BlockSpec((1, _EMBED_DIM, _TOKENS_PER_BATCH),
                         lambda b: (b, 0, 0)),
        ],
        out_specs=[
            pl.BlockSpec((1, _EMBED_DIM, _TOKENS_PER_BATCH),
                         lambda b: (b, 0, 0)),
            pl.BlockSpec((1, 1), lambda b: (0, 0),
                         memory_space=pltpu.SMEM),
        ],
        out_shape=[
            jax.ShapeDtypeStruct((_BATCH, _EMBED_DIM, _TOKENS_PER_BATCH),
                                 jnp.float32),
            jax.ShapeDtypeStruct((1, 1), jnp.float32),
        ],
        compiler_params=pltpu.CompilerParams(
            dimension_semantics=("arbitrary",)),
    )(zq, zr)


def kernel(z, codebook):
    zr = z.reshape(_BATCH, _EMBED_DIM, _TOKENS_PER_BATCH)
    (mi,) = _argmin_call(codebook, zr)
    idx = mi.reshape(_N_TOKENS)
    zq = _sc_gather_kernel()(codebook, idx)
    out3, loss_raw = _assemble_call(zq, zr)
    out = out3.reshape(z.shape)
    loss = (loss_raw * jnp.float32((1.0 + _BETA) / (_N_TOKENS * _EMBED_DIM))
            ).reshape(())
    return out, loss, idx
